# speedup vs baseline: 2.5832x; 2.5832x over previous
import functools

import jax
import jax.numpy as jnp
from jax.experimental import pallas as pl
from jax.experimental.pallas import tpu as pltpu

_NEG = -1e30


def _select_kernel(age_ref, keep_ref, *, m_valid, k_sel):
    a = age_ref[...]
    bits = jax.lax.bitcast_convert_type(a, jnp.int32)
    rows, cols = a.shape
    idx = (jax.lax.broadcasted_iota(jnp.int32, (rows, cols), 0) * cols
           + jax.lax.broadcasted_iota(jnp.int32, (rows, cols), 1))
    valid = idx < m_valid

    def vbody(i, t):
        cand = t + jnp.left_shift(jnp.int32(1), 29 - i)
        cnt = jnp.sum((bits >= cand).astype(jnp.int32))
        return jnp.where(cnt >= k_sel, cand, t)

    thr = jax.lax.fori_loop(0, 30, vbody, jnp.int32(0))

    cnt_gt = jnp.sum((bits > thr).astype(jnp.int32))
    need = k_sel - cnt_gt
    eq = jnp.logical_and(bits == thr, valid)

    def ibody(i, l):
        cand = l + jnp.left_shift(jnp.int32(1), 16 - i)
        cnt = jnp.sum(jnp.logical_and(eq, idx < cand).astype(jnp.int32))
        return jnp.where(cnt < need, cand, l)

    lsel = jax.lax.fori_loop(0, 17, ibody, jnp.int32(0))

    drop = jnp.logical_or(bits > thr, jnp.logical_and(eq, idx <= lsel))
    keep = jnp.logical_and(valid, jnp.logical_not(drop))
    keep_ref[...] = keep.astype(jnp.float32)


def _main_kernel(sx_ref, sy_ref, qx_ref, qy_ref, wb_ref, bb_ref, kw_ref,
                 kb_ref, vw_ref, vb_ref, w1a_ref, w1b_ref, b1_ref, w2_ref,
                 b2_ref, w3_ref, b3_ref, mk_ref, mv_ref, keep_ref,
                 loss_ref, pred_ref, retr_ref,
                 pq_s, qf_s, pk_s, pv_s, m_s, l_s, acc_s,
                 *, nblk, blk, m_valid):
    g = pl.program_id(0)

    @pl.when(g == 0)
    def _prep():
        sf = jnp.dot(sx_ref[...], wb_ref[...],
                     preferred_element_type=jnp.float32) + bb_ref[...]
        qf = jnp.dot(qx_ref[...], wb_ref[...],
                     preferred_element_type=jnp.float32) + bb_ref[...]
        qf_s[...] = qf
        pk_s[...] = jnp.dot(sf, kw_ref[...],
                            preferred_element_type=jnp.float32) + kb_ref[...]
        pq_s[...] = jnp.dot(qf, kw_ref[...],
                            preferred_element_type=jnp.float32) + kb_ref[...]
        pv_s[...] = jnp.dot(sy_ref[...], vw_ref[...],
                            preferred_element_type=jnp.float32) + vb_ref[...]
        m_s[...] = jnp.full_like(m_s[...], _NEG)
        l_s[...] = jnp.zeros_like(l_s[...])
        acc_s[...] = jnp.zeros_like(acc_s[...])

    def flash_update(s, v):
        m_old = m_s[...]
        m_new = jnp.maximum(m_old, jnp.max(s, axis=1, keepdims=True))
        p = jnp.exp(s - m_new)
        corr = jnp.exp(m_old - m_new)
        l_s[...] = l_s[...] * corr + jnp.sum(p, axis=1, keepdims=True)
        acc_s[...] = acc_s[...] * corr + jnp.dot(
            p, v, preferred_element_type=jnp.float32)
        m_s[...] = m_new

    @pl.when(g < nblk)
    def _mem_block():
        q = pq_s[...]
        k = mk_ref[...]
        s = jax.lax.dot_general(q, k, (((1,), (1,)), ((), ())),
                                preferred_element_type=jnp.float32)
        keep = keep_ref[0]
        s = jnp.where(keep > 0.5, s, _NEG)
        row0 = g * blk
        rvalid = (row0 + jax.lax.broadcasted_iota(jnp.int32, (blk, 1), 0)
                  ) < m_valid
        v = jnp.where(rvalid, mv_ref[...], 0.0)
        flash_update(s, v)

    @pl.when(g == nblk)
    def _new_block_and_epilogue():
        q = pq_s[...]
        s = jax.lax.dot_general(q, pk_s[...], (((1,), (1,)), ((), ())),
                                preferred_element_type=jnp.float32)
        flash_update(s, pv_s[...])

        retr = acc_s[...] / l_s[...]
        retr_ref[...] = retr
        h1 = jnp.maximum(
            jnp.dot(qf_s[...], w1a_ref[...],
                    preferred_element_type=jnp.float32)
            + jnp.dot(retr, w1b_ref[...],
                      preferred_element_type=jnp.float32)
            + b1_ref[...], 0.0)
        h2 = jnp.maximum(
            jnp.dot(h1, w2_ref[...], preferred_element_type=jnp.float32)
            + b2_ref[...], 0.0)
        pred = jnp.dot(h2, w3_ref[...],
                       preferred_element_type=jnp.float32) + b3_ref[...]
        pred_ref[...] = pred
        d = pred - qy_ref[...]
        loss_ref[...] = jnp.mean(d * d).reshape(1, 1)


def kernel(support_x, support_y, query_x, query_y, memory_keys, memory_values,
           memory_age, w_base, b_base, key_proj_w, key_proj_b, value_proj_w,
           value_proj_b, ctrl_w1, ctrl_b1, ctrl_w2, ctrl_b2, ctrl_w3,
           ctrl_b3):
    b, din = support_x.shape
    m, kd = memory_keys.shape
    vd = memory_values.shape[1]
    h1d = ctrl_w1.shape[1]
    h2d = ctrl_w2.shape[1]

    blk = 2048
    nblk = -(-m // blk)
    mpad = nblk * blk

    age_p = jnp.pad(memory_age, (0, mpad - m), constant_values=-1.0)
    keep = pl.pallas_call(
        functools.partial(_select_kernel, m_valid=m, k_sel=b),
        out_shape=jax.ShapeDtypeStruct((mpad // 128, 128), jnp.float32),
    )(age_p.reshape(mpad // 128, 128))
    keep_r = keep.reshape(nblk, 1, blk)

    full = lambda shape: pl.BlockSpec(shape, lambda g: (0,) * len(shape))
    memmap = lambda g: (jnp.minimum(g, nblk - 1), 0)

    out = pl.pallas_call(
        functools.partial(_main_kernel, nblk=nblk, blk=blk, m_valid=m),
        grid=(nblk + 1,),
        in_specs=[
            full((b, din)),
            full((b, vd)),
            full((b, din)),
            full((b, 1)),
            full((din, kd)),
            full((1, kd)),
            full((kd, kd)),
            full((1, kd)),
            full((vd, vd)),
            full((1, vd)),
            full((kd, h1d)),
            full((vd, h1d)),
            full((1, h1d)),
            full((h1d, h2d)),
            full((1, h2d)),
            full((h2d, 1)),
            full((1, 1)),
            pl.BlockSpec((blk, kd), memmap),
            pl.BlockSpec((blk, vd), memmap),
            pl.BlockSpec((1, 1, blk),
                         lambda g: (jnp.minimum(g, nblk - 1), 0, 0)),
        ],
        out_specs=[
            full((1, 1)),
            full((b, 1)),
            full((b, vd)),
        ],
        out_shape=[
            jax.ShapeDtypeStruct((1, 1), jnp.float32),
            jax.ShapeDtypeStruct((b, 1), jnp.float32),
            jax.ShapeDtypeStruct((b, vd), jnp.float32),
        ],
        scratch_shapes=[
            pltpu.VMEM((b, kd), jnp.float32),
            pltpu.VMEM((b, kd), jnp.float32),
            pltpu.VMEM((b, kd), jnp.float32),
            pltpu.VMEM((b, vd), jnp.float32),
            pltpu.VMEM((b, 1), jnp.float32),
            pltpu.VMEM((b, 1), jnp.float32),
            pltpu.VMEM((b, vd), jnp.float32),
        ],
    )(support_x, support_y, query_x, query_y, w_base,
      b_base.reshape(1, kd), key_proj_w, key_proj_b.reshape(1, kd),
      value_proj_w, value_proj_b.reshape(1, vd), ctrl_w1[:kd], ctrl_w1[kd:],
      ctrl_b1.reshape(1, h1d), ctrl_w2, ctrl_b2.reshape(1, h2d), ctrl_w3,
      ctrl_b3.reshape(1, 1), memory_keys, memory_values, keep_r)

    loss, pred, retr = out
    return loss.reshape(()), pred, retr


# speedup vs baseline: 2.6628x; 1.0308x over previous
import functools

import jax
import jax.numpy as jnp
from jax.experimental import pallas as pl
from jax.experimental.pallas import tpu as pltpu

_NEG = -1e30
_LOG2E = 1.4426950408889634


def _select_kernel(age_ref, keep_ref, *, m_valid, k_sel):
    a = age_ref[...]
    bits = jax.lax.bitcast_convert_type(a, jnp.int32)
    rows, cols = a.shape
    idx = (jax.lax.broadcasted_iota(jnp.int32, (rows, cols), 0) * cols
           + jax.lax.broadcasted_iota(jnp.int32, (rows, cols), 1))
    valid = idx < m_valid

    def vbody(i, t):
        cand = t + jnp.left_shift(jnp.int32(1), 29 - i)
        cnt = jnp.sum((bits >= cand).astype(jnp.int32))
        return jnp.where(cnt >= k_sel, cand, t)

    thr = jax.lax.fori_loop(0, 30, vbody, jnp.int32(0))

    cnt_gt = jnp.sum((bits > thr).astype(jnp.int32))
    need = k_sel - cnt_gt
    eq = jnp.logical_and(bits == thr, valid)

    def ibody(i, l):
        cand = l + jnp.left_shift(jnp.int32(1), 16 - i)
        cnt = jnp.sum(jnp.logical_and(eq, idx < cand).astype(jnp.int32))
        return jnp.where(cnt < need, cand, l)

    lsel = jax.lax.fori_loop(0, 17, ibody, jnp.int32(0))

    drop = jnp.logical_or(bits > thr, jnp.logical_and(eq, idx <= lsel))
    keep = jnp.logical_and(valid, jnp.logical_not(drop))
    keep_ref[...] = jnp.where(keep, 0.0, _NEG)


def _main_kernel(sx_ref, sy_ref, qx_ref, qy_ref, wb_ref, bb_ref, kw_ref,
                 kb_ref, vw_ref, vb_ref, w1a_ref, w1b_ref, b1_ref, w2_ref,
                 b2_ref, w3_ref, b3_ref, mk_ref, mv_ref, keep_ref,
                 loss_ref, pred_ref, retr_ref,
                 pq_s, qf_s, pk_s, pv_s, m_s, l_s, acc_s,
                 *, nblk, blk, m_valid):
    g = pl.program_id(0)

    @pl.when(g == 0)
    def _prep():
        sf = jnp.dot(sx_ref[...], wb_ref[...],
                     preferred_element_type=jnp.float32) + bb_ref[...]
        qf = jnp.dot(qx_ref[...], wb_ref[...],
                     preferred_element_type=jnp.float32) + bb_ref[...]
        qf_s[...] = qf
        pk_s[...] = jnp.dot(sf, kw_ref[...],
                            preferred_element_type=jnp.float32) + kb_ref[...]
        pq_s[...] = (jnp.dot(qf, kw_ref[...],
                             preferred_element_type=jnp.float32)
                     + kb_ref[...]) * _LOG2E
        pv_s[...] = jnp.dot(sy_ref[...], vw_ref[...],
                            preferred_element_type=jnp.float32) + vb_ref[...]
        m_s[...] = jnp.full_like(m_s[...], _NEG)
        l_s[...] = jnp.zeros_like(l_s[...])
        acc_s[...] = jnp.zeros_like(acc_s[...])

    def flash_update(s, v):
        m_old = m_s[...]
        m_new = jnp.maximum(m_old, jnp.max(s, axis=1, keepdims=True))
        p = jnp.exp2(s - m_new)
        corr = jnp.exp2(m_old - m_new)
        l_s[...] = l_s[...] * corr + jnp.sum(p, axis=1, keepdims=True)
        acc_s[...] = acc_s[...] * corr + jnp.dot(
            p, v, preferred_element_type=jnp.float32)
        m_s[...] = m_new

    @pl.when(g < nblk)
    def _mem_block():
        q = pq_s[...]
        k = mk_ref[...]
        s = jax.lax.dot_general(q, k, (((1,), (1,)), ((), ())),
                                preferred_element_type=jnp.float32)
        flash_update(s + keep_ref[0], mv_ref[...])

    @pl.when(g == nblk)
    def _new_block_and_epilogue():
        q = pq_s[...]
        s = jax.lax.dot_general(q, pk_s[...], (((1,), (1,)), ((), ())),
                                preferred_element_type=jnp.float32)
        flash_update(s, pv_s[...])

        retr = acc_s[...] / l_s[...]
        retr_ref[...] = retr
        h1 = jnp.maximum(
            jnp.dot(qf_s[...], w1a_ref[...],
                    preferred_element_type=jnp.float32)
            + jnp.dot(retr, w1b_ref[...],
                      preferred_element_type=jnp.float32)
            + b1_ref[...], 0.0)
        h2 = jnp.maximum(
            jnp.dot(h1, w2_ref[...], preferred_element_type=jnp.float32)
            + b2_ref[...], 0.0)
        pred = jnp.dot(h2, w3_ref[...],
                       preferred_element_type=jnp.float32) + b3_ref[...]
        pred_ref[...] = pred
        d = pred - qy_ref[...]
        loss_ref[...] = jnp.mean(d * d).reshape(1, 1)


def kernel(support_x, support_y, query_x, query_y, memory_keys, memory_values,
           memory_age, w_base, b_base, key_proj_w, key_proj_b, value_proj_w,
           value_proj_b, ctrl_w1, ctrl_b1, ctrl_w2, ctrl_b2, ctrl_w3,
           ctrl_b3):
    b, din = support_x.shape
    m, kd = memory_keys.shape
    vd = memory_values.shape[1]
    h1d = ctrl_w1.shape[1]
    h2d = ctrl_w2.shape[1]

    blk = 5000
    nblk = -(-m // blk)
    mpad = nblk * blk

    age_p = jnp.pad(memory_age, (0, mpad - m), constant_values=-1.0)
    keep = pl.pallas_call(
        functools.partial(_select_kernel, m_valid=m, k_sel=b),
        out_shape=jax.ShapeDtypeStruct((nblk, blk), jnp.float32),
    )(age_p.reshape(nblk, blk))
    keep_r = keep.reshape(nblk, 1, blk)

    full = lambda shape: pl.BlockSpec(shape, lambda g: (0,) * len(shape))
    memmap = lambda g: (jnp.minimum(g, nblk - 1), 0)

    out = pl.pallas_call(
        functools.partial(_main_kernel, nblk=nblk, blk=blk, m_valid=m),
        grid=(nblk + 1,),
        in_specs=[
            full((b, din)),
            full((b, vd)),
            full((b, din)),
            full((b, 1)),
            full((din, kd)),
            full((1, kd)),
            full((kd, kd)),
            full((1, kd)),
            full((vd, vd)),
            full((1, vd)),
            full((kd, h1d)),
            full((vd, h1d)),
            full((1, h1d)),
            full((h1d, h2d)),
            full((1, h2d)),
            full((h2d, 1)),
            full((1, 1)),
            pl.BlockSpec((blk, kd), memmap),
            pl.BlockSpec((blk, vd), memmap),
            pl.BlockSpec((1, 1, blk),
                         lambda g: (jnp.minimum(g, nblk - 1), 0, 0)),
        ],
        out_specs=[
            full((1, 1)),
            full((b, 1)),
            full((b, vd)),
        ],
        out_shape=[
            jax.ShapeDtypeStruct((1, 1), jnp.float32),
            jax.ShapeDtypeStruct((b, 1), jnp.float32),
            jax.ShapeDtypeStruct((b, vd), jnp.float32),
        ],
        scratch_shapes=[
            pltpu.VMEM((b, kd), jnp.float32),
            pltpu.VMEM((b, kd), jnp.float32),
            pltpu.VMEM((b, kd), jnp.float32),
            pltpu.VMEM((b, vd), jnp.float32),
            pltpu.VMEM((b, 1), jnp.float32),
            pltpu.VMEM((b, 1), jnp.float32),
            pltpu.VMEM((b, vd), jnp.float32),
        ],
    )(support_x, support_y, query_x, query_y, w_base,
      b_base.reshape(1, kd), key_proj_w, key_proj_b.reshape(1, kd),
      value_proj_w, value_proj_b.reshape(1, vd), ctrl_w1[:kd], ctrl_w1[kd:],
      ctrl_b1.reshape(1, h1d), ctrl_w2, ctrl_b2.reshape(1, h2d), ctrl_w3,
      ctrl_b3.reshape(1, 1), memory_keys, memory_values, keep_r)

    loss, pred, retr = out
    return loss.reshape(()), pred, retr
